# baseline (speedup 1.0000x reference)
import functools

import jax
import jax.numpy as jnp
from jax import lax
from jax.experimental import pallas as pl
from jax.experimental.pallas import tpu as pltpu
from jax.experimental.pallas import tpu_sc as plsc

_EPS = 1e-8
_NC, _NS = 2, 16
_NW = _NC * _NS
_INTERPRET = False


def _sc_mesh():
    return plsc.VectorSubcoreMesh(
        core_axis_name="c", subcore_axis_name="s",
        num_cores=_NC, num_subcores=_NS)


def _sc_gather(table, idx2d):
    V, D = table.shape
    B = idx2d.shape[0] * 128
    bw = B // _NW
    kk = bw // 128

    @functools.partial(
        pl.kernel,
        out_type=jax.ShapeDtypeStruct((B, D), jnp.float32),
        mesh=_sc_mesh(),
        scratch_types=[
            pltpu.VMEM((kk, 128), jnp.int32),
            pltpu.VMEM((bw, D), jnp.float32),
            pltpu.SemaphoreType.DMA,
        ],
    )
    def k(table_hbm, idx_hbm, out_hbm, idx_v, rows_v, sem):
        wid = lax.axis_index("s") * _NC + lax.axis_index("c")
        pltpu.sync_copy(idx_hbm.at[pl.ds(wid * kk, kk)], idx_v)
        cps = [pltpu.async_copy(table_hbm.at[idx_v.at[j]],
                                rows_v.at[pl.ds(j * 128, 128)], sem)
               for j in range(kk)]
        for cp in cps:
            cp.wait()
        pltpu.sync_copy(rows_v, out_hbm.at[pl.ds(wid * bw, bw)])

    return k(table, idx2d)


def _sc_scatter_copy(table, idx2d, rows):
    V, D = table.shape
    B = rows.shape[0]
    half = V // _NC
    vw = half // _NS
    jk = B // 128 // _NS

    @functools.partial(
        pl.kernel,
        out_type=jax.ShapeDtypeStruct((V + 8, D), jnp.float32),
        mesh=_sc_mesh(),
        scratch_types=[
            pltpu.VMEM((B // 128, 128), jnp.int32),
            pltpu.VMEM((jk * 128, D), jnp.float32),
            pltpu.SemaphoreType.DMA,
        ],
    )
    def k(t_hbm, i_hbm, r_hbm, o_hbm, idx_v, rows_v, sem):
        cc = lax.axis_index("c")
        ss = lax.axis_index("s")
        hb = cc * half
        lo = hb + ss * vw
        cp = pltpu.async_copy(t_hbm.at[pl.ds(lo, vw)],
                              o_hbm.at[pl.ds(lo, vw)], sem)
        pltpu.sync_copy(i_hbm, idx_v)
        pltpu.sync_copy(r_hbm.at[pl.ds(ss * jk * 128, jk * 128)], rows_v)
        trash = jnp.int32(V) + cc * 4
        for g in range(jk):
            r = ss * jk + g
            for l in range(8):
                v = idx_v[r, pl.ds(l * 16, 16)]
                inb = jnp.logical_and(v >= hb, v < hb + half)
                idx_v[r, pl.ds(l * 16, 16)] = jnp.where(inb, v, trash)
        cp.wait()
        plsc.subcore_barrier()
        scats = []
        for g in range(jk):
            scats.append(pltpu.async_copy(
                rows_v.at[pl.ds(g * 128, 128)],
                o_hbm.at[idx_v.at[ss * jk + g]], sem))
        for sc in scats:
            sc.wait()

    return k(table, idx2d, rows)


_QT, _KT = 1024, 512


def _tc_argmax(unkT, known):
    D, Nm = unkT.shape
    Nk = known.shape[0]
    nq, nk = Nm // _QT, Nk // _KT

    def body(u_ref, kn_ref, dmax_ref, idx_ref):
        kstep = pl.program_id(1)
        u = u_ref[...]
        un = u / (jnp.sqrt(jnp.sum(u * u, axis=0, keepdims=True)) + _EPS)
        kr = kn_ref[...]
        kn = kr / (jnp.sqrt(jnp.sum(kr * kr, axis=1, keepdims=True)) + _EPS)
        sim = lax.dot_general(kn, un, (((1,), (0,)), ((), ())),
                              preferred_element_type=jnp.float32)
        tmax = jnp.max(sim, axis=0)
        rid = lax.broadcasted_iota(jnp.int32, sim.shape, 0)
        targ = jnp.min(jnp.where(sim == tmax[None, :], rid, Nk), axis=0)
        targ = targ + kstep * _KT
        tmax3 = tmax.reshape(1, 1, _QT)
        targ3 = targ.reshape(1, 1, _QT)

        @pl.when(kstep == 0)
        def _():
            dmax_ref[...] = tmax3
            idx_ref[...] = targ3

        @pl.when(kstep != 0)
        def _():
            prev = dmax_ref[...]
            upd = tmax3 > prev
            idx_ref[...] = jnp.where(upd, targ3, idx_ref[...])
            dmax_ref[...] = jnp.where(upd, tmax3, prev)

    return pl.pallas_call(
        body,
        grid=(nq, nk),
        in_specs=[
            pl.BlockSpec((D, _QT), lambda q, k: (0, q)),
            pl.BlockSpec((_KT, D), lambda q, k: (k, 0)),
        ],
        out_specs=[
            pl.BlockSpec((1, 1, _QT), lambda q, k: (q, 0, 0)),
            pl.BlockSpec((1, 1, _QT), lambda q, k: (q, 0, 0)),
        ],
        out_shape=[
            jax.ShapeDtypeStruct((nq, 1, _QT), jnp.float32),
            jax.ShapeDtypeStruct((nq, 1, _QT), jnp.int32),
        ],
        interpret=_INTERPRET,
    )(unkT, known)


def _tc_scan(matched, unk, dmax_col):
    N, D = matched.shape
    CH = 512
    K = 16
    NB = N // K

    def body(m_ref, u_ref, d_ref, g_ref, un_s, dm_s, nd_s,
             m1_s, m2_s, pu_s, pd_s, cw_s, bas_s, mx1_s, mx2_s):
        for cch in range(N // CH):
            sl = pl.ds(cch * CH, CH)
            u = u_ref[sl, :]
            un_s[sl, :] = u / (jnp.sqrt(jnp.sum(u * u, axis=1, keepdims=True))
                               + _EPS)
            dm = m_ref[sl, :] * d_ref[sl, :]
            dm_s[sl, :] = dm
            nd_s[sl, :] = jnp.sum(dm * dm, axis=1, keepdims=True)
        un_s[pl.ds(N, K), :] = jnp.zeros((K, D), jnp.float32)
        dm_s[pl.ds(N, K), :] = jnp.zeros((K, D), jnp.float32)
        bas_s[K + 1:, :] = jnp.zeros((K - 1, D), jnp.float32)
        cw_s[...] = jnp.zeros((K, 2 * K), jnp.float32)
        mx1_s[:, K + 1:] = jnp.zeros((K, K - 1), jnp.float32)
        mx2_s[:, K + 1:] = jnp.zeros((K, K - 1), jnp.float32)

        cdims = (((1,), (1,)), ((), ()))

        def block(base, P, pu, pd, np2, r, M1v, M2v):
            UN = un_s[pl.ds(base, K), :]
            DM = dm_s[pl.ds(base, K), :]
            UNn = un_s[pl.ds(base + K, K), :]
            DMn = dm_s[pl.ds(base + K, K), :]
            M1n = lax.dot_general(UNn, DMn, cdims,
                                  preferred_element_type=jnp.float32)
            M2n = lax.dot_general(DMn, DMn, cdims,
                                  preferred_element_type=jnp.float32)
            mx1_s[:, 0:1] = lax.dot_general(UNn, P, cdims,
                                            preferred_element_type=jnp.float32)
            mx1_s[:, 1:K + 1] = lax.dot_general(UNn, DM, cdims,
                                                preferred_element_type=jnp.float32)
            mx2_s[:, 0:1] = lax.dot_general(DMn, P, cdims,
                                            preferred_element_type=jnp.float32)
            mx2_s[:, 1:K + 1] = lax.dot_general(DMn, DM, cdims,
                                                preferred_element_type=jnp.float32)
            m1_s[...] = M1v
            m2_s[...] = M2v
            pu_s[...] = pu
            pd_s[...] = pd
            Du = [pu_s[t:t + 1, 0:1] for t in range(K)]
            Dd = [pd_s[t:t + 1, 0:1] for t in range(K)]
            dv = [d_ref[pl.ds(base + t, 1), :] for t in range(K)]
            nv = [nd_s[pl.ds(base + t, 1), :] for t in range(K)]
            c = None
            w = [None] * K
            for t in range(K):
                dad = jnp.maximum(Du[t], 0.0) * r
                inv = 1.0 / (dad + dv[t] + _EPS)
                a = dad * inv
                np2 = (dad * dad * np2 + 2.0 * dad * Dd[t] + nv[t]) * (
                    inv * inv)
                r0 = lax.rsqrt(np2 + 1e-36)
                r = r0 - _EPS * r0 * r0
                for tp in range(t + 1, K):
                    g1 = m1_s[tp:tp + 1, t:t + 1]
                    g2 = m2_s[tp:tp + 1, t:t + 1]
                    Du[tp] = a * Du[tp] + inv * g1
                    Dd[tp] = a * Dd[tp] + inv * g2
                c = a if c is None else a * c
                for j in range(t):
                    w[j] = a * w[j]
                w[t] = inv
                cw_s[t:t + 1, 0:1] = c
                for j in range(t + 1):
                    cw_s[t:t + 1, 1 + j:2 + j] = w[j]
            cw7 = cw_s[K - 1:K, :]
            pun = lax.dot_general(mx1_s[...], cw7, cdims,
                                  preferred_element_type=jnp.float32)
            pdn = lax.dot_general(mx2_s[...], cw7, cdims,
                                  preferred_element_type=jnp.float32)
            bas_s[0:1, :] = P
            bas_s[1:K + 1, :] = DM
            GB = lax.dot_general(cw_s[...], bas_s[...],
                                 (((1,), (0,)), ((), ())),
                                 preferred_element_type=jnp.float32)
            g_ref[pl.ds(base, K), :] = GB
            return pun, pdn, np2, r, M1n, M2n

        P0 = m_ref[0:1, :]
        np20 = jnp.sum(P0 * P0, axis=1, keepdims=True)
        r0 = 1.0 / (jnp.sqrt(np20) + _EPS)
        pu0 = lax.dot_general(un_s[pl.ds(0, K), :], P0, cdims,
                              preferred_element_type=jnp.float32)
        pd0 = lax.dot_general(dm_s[pl.ds(0, K), :], P0, cdims,
                              preferred_element_type=jnp.float32)
        M10 = lax.dot_general(un_s[pl.ds(0, K), :], dm_s[pl.ds(0, K), :],
                              cdims, preferred_element_type=jnp.float32)
        M20 = lax.dot_general(dm_s[pl.ds(0, K), :], dm_s[pl.ds(0, K), :],
                              cdims, preferred_element_type=jnp.float32)
        carry0 = block(0, P0, pu0, pd0, np20, r0, M10, M20)

        def lbody(i, carry):
            pu, pd, np2, r, M1v, M2v = carry
            base = i * K
            P = g_ref[pl.ds(base - 1, 1), :]
            return block(base, P, pu, pd, np2, r, M1v, M2v)

        lax.fori_loop(1, NB, lbody, carry0)

    return pl.pallas_call(
        body,
        out_shape=jax.ShapeDtypeStruct((N, D), jnp.float32),
        scratch_shapes=[
            pltpu.VMEM((N + K, D), jnp.float32),
            pltpu.VMEM((N + K, D), jnp.float32),
            pltpu.VMEM((N, 1), jnp.float32),
            pltpu.VMEM((K, K), jnp.float32),
            pltpu.VMEM((K, K), jnp.float32),
            pltpu.VMEM((K, 1), jnp.float32),
            pltpu.VMEM((K, 1), jnp.float32),
            pltpu.VMEM((K, 2 * K), jnp.float32),
            pltpu.VMEM((2 * K, D), jnp.float32),
            pltpu.VMEM((K, 2 * K), jnp.float32),
            pltpu.VMEM((K, 2 * K), jnp.float32),
        ],
        interpret=_INTERPRET,
    )(matched, unk, dmax_col)


def kernel(input, nonmask_point_idx, mask_point_idx):
    x = input
    B, C, H, W = x.shape
    c = C // 2
    HW = H * W
    Nk = nonmask_point_idx.shape[0]
    Nm = mask_point_idx.shape[0]

    former = x[:, :c]
    lf = x[0, c:].reshape(c, HW)
    lf_t = jnp.concatenate(
        [lf.T, jnp.zeros((HW, 128 - c), jnp.float32)], axis=1)

    cat_idx = jnp.concatenate(
        [nonmask_point_idx, mask_point_idx]).reshape(-1, 128)
    g = _sc_gather(lf_t, cat_idx)
    known_t = g[:Nk]
    unk_t = g[Nk:]

    dmax3, idx3 = _tc_argmax(unk_t.T, known_t)
    dmax_col = dmax3.reshape(Nm, 1)
    idx2d = idx3.reshape(-1, 128)

    matched = _sc_gather(known_t, idx2d)
    gens = _tc_scan(matched, unk_t, dmax_col)

    out_t = _sc_scatter_copy(lf_t, mask_point_idx.reshape(-1, 128), gens)
    lf_new = out_t[:HW, :c].T.reshape(1, c, H, W)
    return jnp.concatenate([former, lf_new], axis=1)


# speedup vs baseline: 1.5807x; 1.5807x over previous
import functools

import jax
import jax.numpy as jnp
from jax import lax
from jax.experimental import pallas as pl
from jax.experimental.pallas import tpu as pltpu
from jax.experimental.pallas import tpu_sc as plsc

_EPS = 1e-8
_NC, _NS = 2, 16
_NW = _NC * _NS
_INTERPRET = False


def _sc_mesh():
    return plsc.VectorSubcoreMesh(
        core_axis_name="c", subcore_axis_name="s",
        num_cores=_NC, num_subcores=_NS)


def _sc_gather(table, idx2d):
    V, D = table.shape
    B = idx2d.shape[0] * 128
    bw = B // _NW
    kk = bw // 128

    @functools.partial(
        pl.kernel,
        out_type=jax.ShapeDtypeStruct((B, D), jnp.float32),
        mesh=_sc_mesh(),
        scratch_types=[
            pltpu.VMEM((kk, 128), jnp.int32),
            pltpu.VMEM((bw, D), jnp.float32),
            pltpu.SemaphoreType.DMA,
        ],
    )
    def k(table_hbm, idx_hbm, out_hbm, idx_v, rows_v, sem):
        wid = lax.axis_index("s") * _NC + lax.axis_index("c")
        pltpu.sync_copy(idx_hbm.at[pl.ds(wid * kk, kk)], idx_v)
        cps = [pltpu.async_copy(table_hbm.at[idx_v.at[j]],
                                rows_v.at[pl.ds(j * 128, 128)], sem)
               for j in range(kk)]
        for cp in cps:
            cp.wait()
        pltpu.sync_copy(rows_v, out_hbm.at[pl.ds(wid * bw, bw)])

    return k(table, idx2d)


def _sc_scatter_copy(table, idx2d, rows):
    V, D = table.shape
    B = rows.shape[0]
    half = V // _NC
    vw = half // _NS
    jk = B // 128 // _NS

    @functools.partial(
        pl.kernel,
        out_type=jax.ShapeDtypeStruct((V + 8, D), jnp.float32),
        mesh=_sc_mesh(),
        scratch_types=[
            pltpu.VMEM((vw // 2, D), jnp.float32),
            pltpu.VMEM((B // 128, 128), jnp.int32),
            pltpu.VMEM((jk * 128, D), jnp.float32),
            pltpu.SemaphoreType.DMA,
        ],
    )
    def k(t_hbm, i_hbm, r_hbm, o_hbm, buf_v, idx_v, rows_v, sem):
        cc = lax.axis_index("c")
        ss = lax.axis_index("s")
        hb = cc * half
        lo = hb + ss * vw
        pltpu.sync_copy(i_hbm, idx_v)
        pltpu.sync_copy(r_hbm.at[pl.ds(ss * jk * 128, jk * 128)], rows_v)
        for h in range(2):
            pltpu.sync_copy(t_hbm.at[pl.ds(lo + h * (vw // 2), vw // 2)],
                            buf_v)
            pltpu.sync_copy(buf_v,
                            o_hbm.at[pl.ds(lo + h * (vw // 2), vw // 2)])
        trash = jnp.int32(V) + cc * 4
        for g in range(jk):
            r = ss * jk + g
            for l in range(8):
                v = idx_v[r, pl.ds(l * 16, 16)]
                inb = jnp.logical_and(v >= hb, v < hb + half)
                idx_v[r, pl.ds(l * 16, 16)] = jnp.where(inb, v, trash)
        plsc.subcore_barrier()
        scats = []
        for g in range(jk):
            scats.append(pltpu.async_copy(
                rows_v.at[pl.ds(g * 128, 128)],
                o_hbm.at[idx_v.at[ss * jk + g]], sem))
        for sc in scats:
            sc.wait()

    return k(table, idx2d, rows)


_QT, _KT = 1024, 512


def _tc_argmax(unkT, known):
    D, Nm = unkT.shape
    Nk = known.shape[0]
    nq, nk = Nm // _QT, Nk // _KT

    def body(u_ref, kn_ref, dmax_ref, idx_ref):
        kstep = pl.program_id(1)
        u = u_ref[...]
        un = u / (jnp.sqrt(jnp.sum(u * u, axis=0, keepdims=True)) + _EPS)
        kr = kn_ref[...]
        kn = kr / (jnp.sqrt(jnp.sum(kr * kr, axis=1, keepdims=True)) + _EPS)
        sim = lax.dot_general(kn, un, (((1,), (0,)), ((), ())),
                              preferred_element_type=jnp.float32)
        tmax = jnp.max(sim, axis=0)
        rid = lax.broadcasted_iota(jnp.int32, sim.shape, 0)
        targ = jnp.min(jnp.where(sim == tmax[None, :], rid, Nk), axis=0)
        targ = targ + kstep * _KT
        tmax3 = tmax.reshape(1, 1, _QT)
        targ3 = targ.reshape(1, 1, _QT)

        @pl.when(kstep == 0)
        def _():
            dmax_ref[...] = tmax3
            idx_ref[...] = targ3

        @pl.when(kstep != 0)
        def _():
            prev = dmax_ref[...]
            upd = tmax3 > prev
            idx_ref[...] = jnp.where(upd, targ3, idx_ref[...])
            dmax_ref[...] = jnp.where(upd, tmax3, prev)

    return pl.pallas_call(
        body,
        grid=(nq, nk),
        in_specs=[
            pl.BlockSpec((D, _QT), lambda q, k: (0, q)),
            pl.BlockSpec((_KT, D), lambda q, k: (k, 0)),
        ],
        out_specs=[
            pl.BlockSpec((1, 1, _QT), lambda q, k: (q, 0, 0)),
            pl.BlockSpec((1, 1, _QT), lambda q, k: (q, 0, 0)),
        ],
        out_shape=[
            jax.ShapeDtypeStruct((nq, 1, _QT), jnp.float32),
            jax.ShapeDtypeStruct((nq, 1, _QT), jnp.int32),
        ],
        interpret=_INTERPRET,
    )(unkT, known)


def _tc_scan(matched, unk, dmax_col):
    N, D = matched.shape
    CH = 512
    K = 16
    NB = N // K

    def body(m_ref, u_ref, d_ref, g_ref, un_s, dm_s, nd_s,
             m1_s, m2_s, pu_s, pd_s, cw_s, bas_s, mx1_s, mx2_s):
        for cch in range(N // CH):
            sl = pl.ds(cch * CH, CH)
            u = u_ref[sl, :]
            un_s[sl, :] = u / (jnp.sqrt(jnp.sum(u * u, axis=1, keepdims=True))
                               + _EPS)
            dm = m_ref[sl, :] * d_ref[sl, :]
            dm_s[sl, :] = dm
            nd_s[sl, :] = jnp.sum(dm * dm, axis=1, keepdims=True)
        un_s[pl.ds(N, K), :] = jnp.zeros((K, D), jnp.float32)
        dm_s[pl.ds(N, K), :] = jnp.zeros((K, D), jnp.float32)
        bas_s[K + 1:, :] = jnp.zeros((K - 1, D), jnp.float32)
        cw_s[...] = jnp.zeros((K, 2 * K), jnp.float32)
        mx1_s[:, K + 1:] = jnp.zeros((K, K - 1), jnp.float32)
        mx2_s[:, K + 1:] = jnp.zeros((K, K - 1), jnp.float32)

        cdims = (((1,), (1,)), ((), ()))

        def block(base, P, pu, pd, np2, r, M1v, M2v):
            UN = un_s[pl.ds(base, K), :]
            DM = dm_s[pl.ds(base, K), :]
            UNn = un_s[pl.ds(base + K, K), :]
            DMn = dm_s[pl.ds(base + K, K), :]
            M1n = lax.dot_general(UNn, DMn, cdims,
                                  preferred_element_type=jnp.float32)
            M2n = lax.dot_general(DMn, DMn, cdims,
                                  preferred_element_type=jnp.float32)
            mx1_s[:, 0:1] = lax.dot_general(UNn, P, cdims,
                                            preferred_element_type=jnp.float32)
            mx1_s[:, 1:K + 1] = lax.dot_general(UNn, DM, cdims,
                                                preferred_element_type=jnp.float32)
            mx2_s[:, 0:1] = lax.dot_general(DMn, P, cdims,
                                            preferred_element_type=jnp.float32)
            mx2_s[:, 1:K + 1] = lax.dot_general(DMn, DM, cdims,
                                                preferred_element_type=jnp.float32)
            m1_s[...] = M1v
            m2_s[...] = M2v
            pu_s[...] = pu
            pd_s[...] = pd
            Du = [pu_s[t:t + 1, 0:1] for t in range(K)]
            Dd = [pd_s[t:t + 1, 0:1] for t in range(K)]
            dv = [d_ref[pl.ds(base + t, 1), :] for t in range(K)]
            nv = [nd_s[pl.ds(base + t, 1), :] for t in range(K)]
            c = None
            w = [None] * K
            for t in range(K):
                dad = jnp.maximum(Du[t], 0.0) * r
                inv = 1.0 / (dad + dv[t] + _EPS)
                a = dad * inv
                np2 = (dad * dad * np2 + 2.0 * dad * Dd[t] + nv[t]) * (
                    inv * inv)
                r0 = lax.rsqrt(np2 + 1e-36)
                r = r0 - _EPS * r0 * r0
                for tp in range(t + 1, K):
                    g1 = m1_s[tp:tp + 1, t:t + 1]
                    g2 = m2_s[tp:tp + 1, t:t + 1]
                    Du[tp] = a * Du[tp] + inv * g1
                    Dd[tp] = a * Dd[tp] + inv * g2
                c = a if c is None else a * c
                for j in range(t):
                    w[j] = a * w[j]
                w[t] = inv
                cw_s[t:t + 1, 0:1] = c
                for j in range(t + 1):
                    cw_s[t:t + 1, 1 + j:2 + j] = w[j]
            cw7 = cw_s[K - 1:K, :]
            pun = lax.dot_general(mx1_s[...], cw7, cdims,
                                  preferred_element_type=jnp.float32)
            pdn = lax.dot_general(mx2_s[...], cw7, cdims,
                                  preferred_element_type=jnp.float32)
            bas_s[0:1, :] = P
            bas_s[1:K + 1, :] = DM
            GB = lax.dot_general(cw_s[...], bas_s[...],
                                 (((1,), (0,)), ((), ())),
                                 preferred_element_type=jnp.float32)
            g_ref[pl.ds(base, K), :] = GB
            return pun, pdn, np2, r, M1n, M2n

        P0 = m_ref[0:1, :]
        np20 = jnp.sum(P0 * P0, axis=1, keepdims=True)
        r0 = 1.0 / (jnp.sqrt(np20) + _EPS)
        pu0 = lax.dot_general(un_s[pl.ds(0, K), :], P0, cdims,
                              preferred_element_type=jnp.float32)
        pd0 = lax.dot_general(dm_s[pl.ds(0, K), :], P0, cdims,
                              preferred_element_type=jnp.float32)
        M10 = lax.dot_general(un_s[pl.ds(0, K), :], dm_s[pl.ds(0, K), :],
                              cdims, preferred_element_type=jnp.float32)
        M20 = lax.dot_general(dm_s[pl.ds(0, K), :], dm_s[pl.ds(0, K), :],
                              cdims, preferred_element_type=jnp.float32)
        carry0 = block(0, P0, pu0, pd0, np20, r0, M10, M20)

        def lbody(i, carry):
            pu, pd, np2, r, M1v, M2v = carry
            base = i * K
            P = g_ref[pl.ds(base - 1, 1), :]
            return block(base, P, pu, pd, np2, r, M1v, M2v)

        lax.fori_loop(1, NB, lbody, carry0)

    return pl.pallas_call(
        body,
        out_shape=jax.ShapeDtypeStruct((N, D), jnp.float32),
        scratch_shapes=[
            pltpu.VMEM((N + K, D), jnp.float32),
            pltpu.VMEM((N + K, D), jnp.float32),
            pltpu.VMEM((N, 1), jnp.float32),
            pltpu.VMEM((K, K), jnp.float32),
            pltpu.VMEM((K, K), jnp.float32),
            pltpu.VMEM((K, 1), jnp.float32),
            pltpu.VMEM((K, 1), jnp.float32),
            pltpu.VMEM((K, 2 * K), jnp.float32),
            pltpu.VMEM((2 * K, D), jnp.float32),
            pltpu.VMEM((K, 2 * K), jnp.float32),
            pltpu.VMEM((K, 2 * K), jnp.float32),
        ],
        interpret=_INTERPRET,
    )(matched, unk, dmax_col)


def kernel(input, nonmask_point_idx, mask_point_idx):
    x = input
    B, C, H, W = x.shape
    c = C // 2
    HW = H * W
    Nk = nonmask_point_idx.shape[0]
    Nm = mask_point_idx.shape[0]

    former = x[:, :c]
    lf = x[0, c:].reshape(c, HW)
    lf_t = jnp.concatenate(
        [lf.T, jnp.zeros((HW, 128 - c), jnp.float32)], axis=1)

    cat_idx = jnp.concatenate(
        [nonmask_point_idx, mask_point_idx]).reshape(-1, 128)
    g = _sc_gather(lf_t, cat_idx)
    known_t = g[:Nk]
    unk_t = g[Nk:]

    dmax3, idx3 = _tc_argmax(unk_t.T, known_t)
    dmax_col = dmax3.reshape(Nm, 1)
    idx2d = idx3.reshape(-1, 128)

    matched = _sc_gather(known_t, idx2d)
    gens = _tc_scan(matched, unk_t, dmax_col)

    out_t = _sc_scatter_copy(lf_t, mask_point_idx.reshape(-1, 128), gens)
    lf_new = out_t[:HW, :c].T.reshape(1, c, H, W)
    return jnp.concatenate([former, lf_new], axis=1)


# speedup vs baseline: 1.5913x; 1.0067x over previous
import functools

import jax
import jax.numpy as jnp
from jax import lax
from jax.experimental import pallas as pl
from jax.experimental.pallas import tpu as pltpu
from jax.experimental.pallas import tpu_sc as plsc

_EPS = 1e-8
_NC, _NS = 2, 16
_NW = _NC * _NS
_INTERPRET = False


def _sc_mesh():
    return plsc.VectorSubcoreMesh(
        core_axis_name="c", subcore_axis_name="s",
        num_cores=_NC, num_subcores=_NS)


def _sc_gather(table, idx2d):
    V, D = table.shape
    B = idx2d.shape[0] * 128
    bw = B // _NW
    kk = bw // 128

    @functools.partial(
        pl.kernel,
        out_type=jax.ShapeDtypeStruct((B, D), jnp.float32),
        mesh=_sc_mesh(),
        scratch_types=[
            pltpu.VMEM((kk, 128), jnp.int32),
            pltpu.VMEM((bw, D), jnp.float32),
            pltpu.SemaphoreType.DMA,
        ],
    )
    def k(table_hbm, idx_hbm, out_hbm, idx_v, rows_v, sem):
        wid = lax.axis_index("s") * _NC + lax.axis_index("c")
        pltpu.sync_copy(idx_hbm.at[pl.ds(wid * kk, kk)], idx_v)
        cps = [pltpu.async_copy(table_hbm.at[idx_v.at[j]],
                                rows_v.at[pl.ds(j * 128, 128)], sem)
               for j in range(kk)]
        for cp in cps:
            cp.wait()
        pltpu.sync_copy(rows_v, out_hbm.at[pl.ds(wid * bw, bw)])

    return k(table, idx2d)


def _sc_scatter_copy(table, idx2d, rows):
    V, D = table.shape
    B = rows.shape[0]
    half = V // _NC
    vw = half // _NS
    jk = B // 128 // _NS

    @functools.partial(
        pl.kernel,
        out_type=jax.ShapeDtypeStruct((V + 8, D), jnp.float32),
        mesh=_sc_mesh(),
        scratch_types=[
            pltpu.VMEM((vw // 2, D), jnp.float32),
            pltpu.VMEM((B // 128, 128), jnp.int32),
            pltpu.VMEM((jk * 128, D), jnp.float32),
            pltpu.SemaphoreType.DMA,
        ],
    )
    def k(t_hbm, i_hbm, r_hbm, o_hbm, buf_v, idx_v, rows_v, sem):
        cc = lax.axis_index("c")
        ss = lax.axis_index("s")
        hb = cc * half
        lo = hb + ss * vw
        pltpu.sync_copy(i_hbm, idx_v)
        pltpu.sync_copy(r_hbm.at[pl.ds(ss * jk * 128, jk * 128)], rows_v)
        for h in range(2):
            pltpu.sync_copy(t_hbm.at[pl.ds(lo + h * (vw // 2), vw // 2)],
                            buf_v)
            pltpu.sync_copy(buf_v,
                            o_hbm.at[pl.ds(lo + h * (vw // 2), vw // 2)])
        trash = jnp.int32(V) + cc * 4
        for g in range(jk):
            r = ss * jk + g
            for l in range(8):
                v = idx_v[r, pl.ds(l * 16, 16)]
                inb = jnp.logical_and(v >= hb, v < hb + half)
                idx_v[r, pl.ds(l * 16, 16)] = jnp.where(inb, v, trash)
        plsc.subcore_barrier()
        scats = []
        for g in range(jk):
            scats.append(pltpu.async_copy(
                rows_v.at[pl.ds(g * 128, 128)],
                o_hbm.at[idx_v.at[ss * jk + g]], sem))
        for sc in scats:
            sc.wait()

    return k(table, idx2d, rows)


_QT, _KT = 1024, 512


def _tc_argmax(unkT, known):
    D, Nm = unkT.shape
    Nk = known.shape[0]
    nq, nk = Nm // _QT, Nk // _KT

    def body(u_ref, kn_ref, dmax_ref, idx_ref, un_s, kn_s,
             dm_acc, ix_acc):
        kq = pl.program_id(0)
        qq = pl.program_id(1)
        qsl = pl.ds(qq * _QT, _QT)

        @pl.when(kq == 0)
        def _():
            u = u_ref[:, qsl]
            un_s[:, qsl] = u / (jnp.sqrt(jnp.sum(u * u, axis=0,
                                                 keepdims=True)) + _EPS)

        @pl.when(qq == 0)
        def _():
            kr = kn_ref[...]
            kn_s[...] = kr / (jnp.sqrt(jnp.sum(kr * kr, axis=1,
                                               keepdims=True)) + _EPS)

        sim = lax.dot_general(kn_s[...], un_s[:, qsl], (((1,), (0,)), ((), ())),
                              preferred_element_type=jnp.float32)
        tmax = jnp.max(sim, axis=0).reshape(1, _QT)
        rid = lax.broadcasted_iota(jnp.int32, sim.shape, 0)
        targ = jnp.min(jnp.where(sim == tmax, rid, Nk), axis=0)
        targ = (targ + kq * _KT).reshape(1, _QT)

        @pl.when(kq == 0)
        def _():
            dm_acc[:, qsl] = tmax
            ix_acc[:, qsl] = targ

        @pl.when(kq != 0)
        def _():
            prev = dm_acc[:, qsl]
            upd = tmax > prev
            ix_acc[:, qsl] = jnp.where(upd, targ, ix_acc[:, qsl])
            dm_acc[:, qsl] = jnp.where(upd, tmax, prev)

        @pl.when(kq == nk - 1)
        def _():
            dmax_ref[...] = dm_acc[:, qsl].reshape(1, 1, _QT)
            idx_ref[...] = ix_acc[:, qsl].reshape(1, 1, _QT)

    return pl.pallas_call(
        body,
        grid=(nk, nq),
        in_specs=[
            pl.BlockSpec((D, Nm), lambda k, q: (0, 0)),
            pl.BlockSpec((_KT, D), lambda k, q: (k, 0)),
        ],
        out_specs=[
            pl.BlockSpec((1, 1, _QT), lambda k, q: (q, 0, 0)),
            pl.BlockSpec((1, 1, _QT), lambda k, q: (q, 0, 0)),
        ],
        out_shape=[
            jax.ShapeDtypeStruct((nq, 1, _QT), jnp.float32),
            jax.ShapeDtypeStruct((nq, 1, _QT), jnp.int32),
        ],
        scratch_shapes=[
            pltpu.VMEM((D, Nm), jnp.float32),
            pltpu.VMEM((_KT, D), jnp.float32),
            pltpu.VMEM((1, Nm), jnp.float32),
            pltpu.VMEM((1, Nm), jnp.int32),
        ],
        interpret=_INTERPRET,
    )(unkT, known)


def _tc_scan(matched, unk, dmax_col):
    N, D = matched.shape
    CH = 512
    K = 16
    NB = N // K

    def body(m_ref, u_ref, d_ref, g_ref, un_s, dm_s, nd_s,
             m1_s, m2_s, pu_s, pd_s, cw_s, bas_s, mx1_s, mx2_s):
        for cch in range(N // CH):
            sl = pl.ds(cch * CH, CH)
            u = u_ref[sl, :]
            un_s[sl, :] = u / (jnp.sqrt(jnp.sum(u * u, axis=1, keepdims=True))
                               + _EPS)
            dm = m_ref[sl, :] * d_ref[sl, :]
            dm_s[sl, :] = dm
            nd_s[sl, :] = jnp.sum(dm * dm, axis=1, keepdims=True)
        un_s[pl.ds(N, K), :] = jnp.zeros((K, D), jnp.float32)
        dm_s[pl.ds(N, K), :] = jnp.zeros((K, D), jnp.float32)
        bas_s[K + 1:, :] = jnp.zeros((K - 1, D), jnp.float32)
        cw_s[...] = jnp.zeros((K, 2 * K), jnp.float32)
        mx1_s[:, K + 1:] = jnp.zeros((K, K - 1), jnp.float32)
        mx2_s[:, K + 1:] = jnp.zeros((K, K - 1), jnp.float32)

        cdims = (((1,), (1,)), ((), ()))

        def block(base, P, pu, pd, np2, r, M1v, M2v):
            UN = un_s[pl.ds(base, K), :]
            DM = dm_s[pl.ds(base, K), :]
            UNn = un_s[pl.ds(base + K, K), :]
            DMn = dm_s[pl.ds(base + K, K), :]
            M1n = lax.dot_general(UNn, DMn, cdims,
                                  preferred_element_type=jnp.float32)
            M2n = lax.dot_general(DMn, DMn, cdims,
                                  preferred_element_type=jnp.float32)
            mx1_s[:, 0:1] = lax.dot_general(UNn, P, cdims,
                                            preferred_element_type=jnp.float32)
            mx1_s[:, 1:K + 1] = lax.dot_general(UNn, DM, cdims,
                                                preferred_element_type=jnp.float32)
            mx2_s[:, 0:1] = lax.dot_general(DMn, P, cdims,
                                            preferred_element_type=jnp.float32)
            mx2_s[:, 1:K + 1] = lax.dot_general(DMn, DM, cdims,
                                                preferred_element_type=jnp.float32)
            m1_s[...] = M1v
            m2_s[...] = M2v
            pu_s[...] = pu
            pd_s[...] = pd
            Du = [pu_s[t:t + 1, 0:1] for t in range(K)]
            Dd = [pd_s[t:t + 1, 0:1] for t in range(K)]
            dv = [d_ref[pl.ds(base + t, 1), :] for t in range(K)]
            nv = [nd_s[pl.ds(base + t, 1), :] for t in range(K)]
            c = None
            w = [None] * K
            for t in range(K):
                dad = jnp.maximum(Du[t], 0.0) * r
                inv = 1.0 / (dad + dv[t] + _EPS)
                a = dad * inv
                np2 = (dad * dad * np2 + 2.0 * dad * Dd[t] + nv[t]) * (
                    inv * inv)
                r0 = lax.rsqrt(np2 + 1e-36)
                r = r0 - _EPS * r0 * r0
                for tp in range(t + 1, K):
                    g1 = m1_s[tp:tp + 1, t:t + 1]
                    g2 = m2_s[tp:tp + 1, t:t + 1]
                    Du[tp] = a * Du[tp] + inv * g1
                    Dd[tp] = a * Dd[tp] + inv * g2
                c = a if c is None else a * c
                for j in range(t):
                    w[j] = a * w[j]
                w[t] = inv
                cw_s[t:t + 1, 0:1] = c
                for j in range(t + 1):
                    cw_s[t:t + 1, 1 + j:2 + j] = w[j]
            cw7 = cw_s[K - 1:K, :]
            pun = lax.dot_general(mx1_s[...], cw7, cdims,
                                  preferred_element_type=jnp.float32)
            pdn = lax.dot_general(mx2_s[...], cw7, cdims,
                                  preferred_element_type=jnp.float32)
            bas_s[0:1, :] = P
            bas_s[1:K + 1, :] = DM
            GB = lax.dot_general(cw_s[...], bas_s[...],
                                 (((1,), (0,)), ((), ())),
                                 preferred_element_type=jnp.float32)
            g_ref[pl.ds(base, K), :] = GB
            return pun, pdn, np2, r, M1n, M2n

        P0 = m_ref[0:1, :]
        np20 = jnp.sum(P0 * P0, axis=1, keepdims=True)
        r0 = 1.0 / (jnp.sqrt(np20) + _EPS)
        pu0 = lax.dot_general(un_s[pl.ds(0, K), :], P0, cdims,
                              preferred_element_type=jnp.float32)
        pd0 = lax.dot_general(dm_s[pl.ds(0, K), :], P0, cdims,
                              preferred_element_type=jnp.float32)
        M10 = lax.dot_general(un_s[pl.ds(0, K), :], dm_s[pl.ds(0, K), :],
                              cdims, preferred_element_type=jnp.float32)
        M20 = lax.dot_general(dm_s[pl.ds(0, K), :], dm_s[pl.ds(0, K), :],
                              cdims, preferred_element_type=jnp.float32)
        carry0 = block(0, P0, pu0, pd0, np20, r0, M10, M20)

        def lbody(i, carry):
            pu, pd, np2, r, M1v, M2v = carry
            base = i * K
            P = g_ref[pl.ds(base - 1, 1), :]
            return block(base, P, pu, pd, np2, r, M1v, M2v)

        lax.fori_loop(1, NB, lbody, carry0)

    return pl.pallas_call(
        body,
        out_shape=jax.ShapeDtypeStruct((N, D), jnp.float32),
        scratch_shapes=[
            pltpu.VMEM((N + K, D), jnp.float32),
            pltpu.VMEM((N + K, D), jnp.float32),
            pltpu.VMEM((N, 1), jnp.float32),
            pltpu.VMEM((K, K), jnp.float32),
            pltpu.VMEM((K, K), jnp.float32),
            pltpu.VMEM((K, 1), jnp.float32),
            pltpu.VMEM((K, 1), jnp.float32),
            pltpu.VMEM((K, 2 * K), jnp.float32),
            pltpu.VMEM((2 * K, D), jnp.float32),
            pltpu.VMEM((K, 2 * K), jnp.float32),
            pltpu.VMEM((K, 2 * K), jnp.float32),
        ],
        interpret=_INTERPRET,
    )(matched, unk, dmax_col)


def kernel(input, nonmask_point_idx, mask_point_idx):
    x = input
    B, C, H, W = x.shape
    c = C // 2
    HW = H * W
    Nk = nonmask_point_idx.shape[0]
    Nm = mask_point_idx.shape[0]

    former = x[:, :c]
    lf = x[0, c:].reshape(c, HW)
    lf_t = jnp.concatenate(
        [lf.T, jnp.zeros((HW, 128 - c), jnp.float32)], axis=1)

    cat_idx = jnp.concatenate(
        [nonmask_point_idx, mask_point_idx]).reshape(-1, 128)
    g = _sc_gather(lf_t, cat_idx)
    known_t = g[:Nk]
    unk_t = g[Nk:]

    dmax3, idx3 = _tc_argmax(unk_t.T, known_t)
    dmax_col = dmax3.reshape(Nm, 1)
    idx2d = idx3.reshape(-1, 128)

    matched = _sc_gather(known_t, idx2d)
    gens = _tc_scan(matched, unk_t, dmax_col)

    out_t = _sc_scatter_copy(lf_t, mask_point_idx.reshape(-1, 128), gens)
    lf_new = out_t[:HW, :c].T.reshape(1, c, H, W)
    return jnp.concatenate([former, lf_new], axis=1)


# speedup vs baseline: 1.5914x; 1.0000x over previous
import functools

import jax
import jax.numpy as jnp
from jax import lax
from jax.experimental import pallas as pl
from jax.experimental.pallas import tpu as pltpu
from jax.experimental.pallas import tpu_sc as plsc

_EPS = 1e-8
_NC, _NS = 2, 16
_NW = _NC * _NS


def _sc_mesh():
    return plsc.VectorSubcoreMesh(
        core_axis_name="c", subcore_axis_name="s",
        num_cores=_NC, num_subcores=_NS)


def _sc_gather(table, idx2d):
    V, D = table.shape
    B = idx2d.shape[0] * 128
    bw = B // _NW
    kk = bw // 128

    @functools.partial(
        pl.kernel,
        out_type=jax.ShapeDtypeStruct((B, D), jnp.float32),
        mesh=_sc_mesh(),
        scratch_types=[
            pltpu.VMEM((kk, 128), jnp.int32),
            pltpu.VMEM((bw, D), jnp.float32),
            pltpu.SemaphoreType.DMA,
        ],
    )
    def k(table_hbm, idx_hbm, out_hbm, idx_v, rows_v, sem):
        wid = lax.axis_index("s") * _NC + lax.axis_index("c")
        pltpu.sync_copy(idx_hbm.at[pl.ds(wid * kk, kk)], idx_v)
        cps = [pltpu.async_copy(table_hbm.at[idx_v.at[j]],
                                rows_v.at[pl.ds(j * 128, 128)], sem)
               for j in range(kk)]
        for cp in cps:
            cp.wait()
        pltpu.sync_copy(rows_v, out_hbm.at[pl.ds(wid * bw, bw)])

    return k(table, idx2d)


def _sc_scatter_copy(table, idx2d, rows):
    V, D = table.shape
    B = rows.shape[0]
    half = V // _NC
    vw = half // _NS
    jk = B // 128 // _NS

    @functools.partial(
        pl.kernel,
        out_type=jax.ShapeDtypeStruct((V + 8, D), jnp.float32),
        mesh=_sc_mesh(),
        scratch_types=[
            pltpu.VMEM((vw // 2, D), jnp.float32),
            pltpu.VMEM((B // 128, 128), jnp.int32),
            pltpu.VMEM((jk * 128, D), jnp.float32),
            pltpu.SemaphoreType.DMA,
        ],
    )
    def k(t_hbm, i_hbm, r_hbm, o_hbm, buf_v, idx_v, rows_v, sem):
        cc = lax.axis_index("c")
        ss = lax.axis_index("s")
        hb = cc * half
        lo = hb + ss * vw
        pltpu.sync_copy(i_hbm, idx_v)
        pltpu.sync_copy(r_hbm.at[pl.ds(ss * jk * 128, jk * 128)], rows_v)
        for h in range(2):
            pltpu.sync_copy(t_hbm.at[pl.ds(lo + h * (vw // 2), vw // 2)],
                            buf_v)
            pltpu.sync_copy(buf_v,
                            o_hbm.at[pl.ds(lo + h * (vw // 2), vw // 2)])
        trash = jnp.int32(V) + cc * 4
        for g in range(jk):
            r = ss * jk + g
            for l in range(8):
                v = idx_v[r, pl.ds(l * 16, 16)]
                inb = jnp.logical_and(v >= hb, v < hb + half)
                idx_v[r, pl.ds(l * 16, 16)] = jnp.where(inb, v, trash)
        plsc.subcore_barrier()
        scats = []
        for g in range(jk):
            scats.append(pltpu.async_copy(
                rows_v.at[pl.ds(g * 128, 128)],
                o_hbm.at[idx_v.at[ss * jk + g]], sem))
        for sc in scats:
            sc.wait()

    return k(table, idx2d, rows)


_QT, _KT = 1024, 512


def _tc_argmax(unkT, known):
    D, Nm = unkT.shape
    Nk = known.shape[0]
    nq, nk = Nm // _QT, Nk // _KT

    def body(u_ref, kn_ref, dmax_ref, idx_ref, un_s, kn_s,
             dm_acc, ix_acc):
        kq = pl.program_id(0)
        qq = pl.program_id(1)
        qsl = pl.ds(qq * _QT, _QT)

        @pl.when(kq == 0)
        def _():
            u = u_ref[:, qsl]
            un_s[:, qsl] = u / (jnp.sqrt(jnp.sum(u * u, axis=0,
                                                 keepdims=True)) + _EPS)

        @pl.when(qq == 0)
        def _():
            kr = kn_ref[...]
            kn_s[...] = kr / (jnp.sqrt(jnp.sum(kr * kr, axis=1,
                                               keepdims=True)) + _EPS)

        sim = lax.dot_general(kn_s[...], un_s[:, qsl], (((1,), (0,)), ((), ())),
                              preferred_element_type=jnp.float32)
        tmax = jnp.max(sim, axis=0).reshape(1, _QT)
        rid = lax.broadcasted_iota(jnp.int32, sim.shape, 0)
        targ = jnp.min(jnp.where(sim == tmax, rid, Nk), axis=0)
        targ = (targ + kq * _KT).reshape(1, _QT)

        @pl.when(kq == 0)
        def _():
            dm_acc[:, qsl] = tmax
            ix_acc[:, qsl] = targ

        @pl.when(kq != 0)
        def _():
            prev = dm_acc[:, qsl]
            upd = tmax > prev
            ix_acc[:, qsl] = jnp.where(upd, targ, ix_acc[:, qsl])
            dm_acc[:, qsl] = jnp.where(upd, tmax, prev)

        @pl.when(kq == nk - 1)
        def _():
            dmax_ref[...] = dm_acc[:, qsl].reshape(1, 1, _QT)
            idx_ref[...] = ix_acc[:, qsl].reshape(1, 1, _QT)

    return pl.pallas_call(
        body,
        grid=(nk, nq),
        in_specs=[
            pl.BlockSpec((D, Nm), lambda k, q: (0, 0)),
            pl.BlockSpec((_KT, D), lambda k, q: (k, 0)),
        ],
        out_specs=[
            pl.BlockSpec((1, 1, _QT), lambda k, q: (q, 0, 0)),
            pl.BlockSpec((1, 1, _QT), lambda k, q: (q, 0, 0)),
        ],
        out_shape=[
            jax.ShapeDtypeStruct((nq, 1, _QT), jnp.float32),
            jax.ShapeDtypeStruct((nq, 1, _QT), jnp.int32),
        ],
        scratch_shapes=[
            pltpu.VMEM((D, Nm), jnp.float32),
            pltpu.VMEM((_KT, D), jnp.float32),
            pltpu.VMEM((1, Nm), jnp.float32),
            pltpu.VMEM((1, Nm), jnp.int32),
        ],
    )(unkT, known)


def _tc_scan(matched, unk, dmax_col):
    N, D = matched.shape
    CH = 512
    K = 16
    NB = N // K

    def body(m_ref, u_ref, d_ref, g_ref, un_s, dm_s, nd_s,
             m1_s, m2_s, pu_s, pd_s, cw_s, bas_s, mx1_s, mx2_s):
        for cch in range(N // CH):
            sl = pl.ds(cch * CH, CH)
            u = u_ref[sl, :]
            un_s[sl, :] = u / (jnp.sqrt(jnp.sum(u * u, axis=1, keepdims=True))
                               + _EPS)
            dm = m_ref[sl, :] * d_ref[sl, :]
            dm_s[sl, :] = dm
            nd_s[sl, :] = jnp.sum(dm * dm, axis=1, keepdims=True)
        un_s[pl.ds(N, K), :] = jnp.zeros((K, D), jnp.float32)
        dm_s[pl.ds(N, K), :] = jnp.zeros((K, D), jnp.float32)
        bas_s[K + 1:, :] = jnp.zeros((K - 1, D), jnp.float32)
        cw_s[...] = jnp.zeros((K, 2 * K), jnp.float32)
        mx1_s[:, K + 1:] = jnp.zeros((K, K - 1), jnp.float32)
        mx2_s[:, K + 1:] = jnp.zeros((K, K - 1), jnp.float32)

        cdims = (((1,), (1,)), ((), ()))

        def block(base, P, pu, pd, np2, r, M1v, M2v):
            UN = un_s[pl.ds(base, K), :]
            DM = dm_s[pl.ds(base, K), :]
            UNn = un_s[pl.ds(base + K, K), :]
            DMn = dm_s[pl.ds(base + K, K), :]
            M1n = lax.dot_general(UNn, DMn, cdims,
                                  preferred_element_type=jnp.float32)
            M2n = lax.dot_general(DMn, DMn, cdims,
                                  preferred_element_type=jnp.float32)
            mx1_s[:, 0:1] = lax.dot_general(UNn, P, cdims,
                                            preferred_element_type=jnp.float32)
            mx1_s[:, 1:K + 1] = lax.dot_general(UNn, DM, cdims,
                                                preferred_element_type=jnp.float32)
            mx2_s[:, 0:1] = lax.dot_general(DMn, P, cdims,
                                            preferred_element_type=jnp.float32)
            mx2_s[:, 1:K + 1] = lax.dot_general(DMn, DM, cdims,
                                                preferred_element_type=jnp.float32)
            m1_s[...] = M1v
            m2_s[...] = M2v
            pu_s[...] = pu
            pd_s[...] = pd
            Du = [pu_s[t:t + 1, 0:1] for t in range(K)]
            Dd = [pd_s[t:t + 1, 0:1] for t in range(K)]
            dv = [d_ref[pl.ds(base + t, 1), :] for t in range(K)]
            nv = [nd_s[pl.ds(base + t, 1), :] for t in range(K)]
            c = None
            w = [None] * K
            for t in range(K):
                dad = jnp.maximum(Du[t], 0.0) * r
                inv = 1.0 / (dad + dv[t] + _EPS)
                a = dad * inv
                np2 = (dad * dad * np2 + 2.0 * dad * Dd[t] + nv[t]) * (
                    inv * inv)
                r0 = lax.rsqrt(np2 + 1e-36)
                r = r0 - _EPS * r0 * r0
                for tp in range(t + 1, K):
                    g1 = m1_s[tp:tp + 1, t:t + 1]
                    g2 = m2_s[tp:tp + 1, t:t + 1]
                    Du[tp] = a * Du[tp] + inv * g1
                    Dd[tp] = a * Dd[tp] + inv * g2
                c = a if c is None else a * c
                for j in range(t):
                    w[j] = a * w[j]
                w[t] = inv
                cw_s[t:t + 1, 0:1] = c
                for j in range(t + 1):
                    cw_s[t:t + 1, 1 + j:2 + j] = w[j]
            cw7 = cw_s[K - 1:K, :]
            pun = lax.dot_general(mx1_s[...], cw7, cdims,
                                  preferred_element_type=jnp.float32)
            pdn = lax.dot_general(mx2_s[...], cw7, cdims,
                                  preferred_element_type=jnp.float32)
            bas_s[0:1, :] = P
            bas_s[1:K + 1, :] = DM
            GB = lax.dot_general(cw_s[...], bas_s[...],
                                 (((1,), (0,)), ((), ())),
                                 preferred_element_type=jnp.float32)
            g_ref[pl.ds(base, K), :] = GB
            return pun, pdn, np2, r, M1n, M2n

        P0 = m_ref[0:1, :]
        np20 = jnp.sum(P0 * P0, axis=1, keepdims=True)
        r0 = 1.0 / (jnp.sqrt(np20) + _EPS)
        pu0 = lax.dot_general(un_s[pl.ds(0, K), :], P0, cdims,
                              preferred_element_type=jnp.float32)
        pd0 = lax.dot_general(dm_s[pl.ds(0, K), :], P0, cdims,
                              preferred_element_type=jnp.float32)
        M10 = lax.dot_general(un_s[pl.ds(0, K), :], dm_s[pl.ds(0, K), :],
                              cdims, preferred_element_type=jnp.float32)
        M20 = lax.dot_general(dm_s[pl.ds(0, K), :], dm_s[pl.ds(0, K), :],
                              cdims, preferred_element_type=jnp.float32)
        carry0 = block(0, P0, pu0, pd0, np20, r0, M10, M20)

        def lbody(i, carry):
            pu, pd, np2, r, M1v, M2v = carry
            base = i * K
            P = g_ref[pl.ds(base - 1, 1), :]
            return block(base, P, pu, pd, np2, r, M1v, M2v)

        lax.fori_loop(1, NB, lbody, carry0)

    return pl.pallas_call(
        body,
        out_shape=jax.ShapeDtypeStruct((N, D), jnp.float32),
        scratch_shapes=[
            pltpu.VMEM((N + K, D), jnp.float32),
            pltpu.VMEM((N + K, D), jnp.float32),
            pltpu.VMEM((N, 1), jnp.float32),
            pltpu.VMEM((K, K), jnp.float32),
            pltpu.VMEM((K, K), jnp.float32),
            pltpu.VMEM((K, 1), jnp.float32),
            pltpu.VMEM((K, 1), jnp.float32),
            pltpu.VMEM((K, 2 * K), jnp.float32),
            pltpu.VMEM((2 * K, D), jnp.float32),
            pltpu.VMEM((K, 2 * K), jnp.float32),
            pltpu.VMEM((K, 2 * K), jnp.float32),
        ],
    )(matched, unk, dmax_col)


def kernel(input, nonmask_point_idx, mask_point_idx):
    x = input
    B, C, H, W = x.shape
    c = C // 2
    HW = H * W
    Nk = nonmask_point_idx.shape[0]
    Nm = mask_point_idx.shape[0]

    former = x[:, :c]
    lf = x[0, c:].reshape(c, HW)
    lf_t = jnp.concatenate(
        [lf.T, jnp.zeros((HW, 128 - c), jnp.float32)], axis=1)

    cat_idx = jnp.concatenate(
        [nonmask_point_idx, mask_point_idx]).reshape(-1, 128)
    g = _sc_gather(lf_t, cat_idx)
    known_t = g[:Nk]
    unk_t = g[Nk:]

    dmax3, idx3 = _tc_argmax(unk_t.T, known_t)
    dmax_col = dmax3.reshape(Nm, 1)
    idx2d = idx3.reshape(-1, 128)

    matched = _sc_gather(known_t, idx2d)
    gens = _tc_scan(matched, unk_t, dmax_col)

    out_t = _sc_scatter_copy(lf_t, mask_point_idx.reshape(-1, 128), gens)
    lf_new = out_t[:HW, :c].T.reshape(1, c, H, W)
    return jnp.concatenate([former, lf_new], axis=1)
